# Initial kernel scaffold; baseline (speedup 1.0000x reference)
import functools

import jax
import jax.numpy as jnp
from jax import lax
from jax.experimental import pallas as pl
from jax.experimental.pallas import tpu as pltpu
from jax.experimental.pallas import tpu_sc as plsc

_N_USERS = 25000
_N_ITEMS = 25000
_N = _N_USERS + _N_ITEMS
_H = 32
_E = 800000
_C = 128
_CHUNKS = _E // _C
_NSUB = 16
_FULL = _CHUNKS // _NSUB
_REM = _CHUNKS - _FULL * _NSUB
_RPT = _N // _NSUB
_WB = 125


def _layer_body(src_hbm, dst_hbm, vals_hbm, elo_hbm, ehi_hbm,
                outlo_hbm, outhi_hbm,
                acc, src_v, dst_v, vals_v, rows_v, zb, sem):
    cid = lax.axis_index("c")
    sid = lax.axis_index("s")

    zeros16 = jnp.zeros((16,), jnp.float32)

    def zb_zero(i, carry):
        zb[i, pl.ds(0, 16)] = zeros16
        zb[i, pl.ds(16, 16)] = zeros16
        return carry

    lax.fori_loop(0, _WB, zb_zero, 0)

    def acc_zero(j, carry):
        pltpu.sync_copy(zb, acc.at[pl.ds(sid * _RPT + j * _WB, _WB)])
        return carry

    lax.fori_loop(0, _RPT // _WB, acc_zero, 0)
    plsc.subcore_barrier()

    def run_half(e_hbm, out_hbm):
        nchunks = _FULL + jnp.where(sid < _REM, 1, 0)

        def chunk_body(i, carry):
            base = (i * _NSUB + sid) * _C
            pltpu.sync_copy(src_hbm.at[pl.ds(base, _C)], src_v)
            pltpu.sync_copy(dst_hbm.at[pl.ds(base, _C)], dst_v)
            pltpu.sync_copy(vals_hbm.at[pl.ds(base, _C)], vals_v)
            pltpu.async_copy(e_hbm.at[src_v], rows_v, sem).wait()
            for g in range(_C // 16):
                vals16 = vals_v[pl.ds(g * 16, 16)]
                for r in range(16):
                    v16 = jnp.take(vals16, jnp.full((16,), r, jnp.int32),
                                   mode="promise_in_bounds")
                    row = g * 16 + r
                    rows_v[row, pl.ds(0, 16)] = rows_v[row, pl.ds(0, 16)] * v16
                    rows_v[row, pl.ds(16, 16)] = rows_v[row, pl.ds(16, 16)] * v16
            pltpu.sync_copy(rows_v, acc.at[dst_v], add=True)
            return carry

        lax.fori_loop(0, nchunks, chunk_body, 0)
        plsc.subcore_barrier()

        def wb(j, carry):
            off = sid * _RPT + j * _WB
            pltpu.sync_copy(acc.at[pl.ds(off, _WB)], zb)
            pltpu.sync_copy(zb, out_hbm.at[pl.ds(off, _WB)])
            return carry

        lax.fori_loop(0, _RPT // _WB, wb, 0)

    @pl.when(cid == 0)
    def _():
        run_half(elo_hbm, outlo_hbm)

    @pl.when(cid == 1)
    def _():
        run_half(ehi_hbm, outhi_hbm)


@functools.cache
def _make_layer():
    mesh = plsc.VectorSubcoreMesh(core_axis_name="c", subcore_axis_name="s")
    return pl.kernel(
        _layer_body,
        out_type=[jax.ShapeDtypeStruct((_N, _H), jnp.float32)] * 2,
        mesh=mesh,
        scratch_types=[
            pltpu.VMEM_SHARED((_N, _H), jnp.float32),
            pltpu.VMEM((_C,), jnp.int32),
            pltpu.VMEM((_C,), jnp.int32),
            pltpu.VMEM((_C,), jnp.float32),
            pltpu.VMEM((_C, _H), jnp.float32),
            pltpu.VMEM((_WB, _H), jnp.float32),
            pltpu.SemaphoreType.DMA,
        ],
    )


def _mean_body(a0, a1, a2, a3, b0, b1, b2, b3, olo, ohi):
    olo[...] = (a0[...] + a1[...] + a2[...] + a3[...]) * 0.25
    ohi[...] = (b0[...] + b1[...] + b2[...] + b3[...]) * 0.25


_BLK = 400


@functools.cache
def _make_mean():
    spec = pl.BlockSpec((_BLK, _H), lambda i: (i, 0))
    return pl.pallas_call(
        _mean_body,
        grid=(_N // _BLK,),
        in_specs=[spec] * 8,
        out_specs=[spec] * 2,
        out_shape=[jax.ShapeDtypeStruct((_N, _H), jnp.float32)] * 2,
    )


def kernel(adj_indices, adj_values, user_emb, item_emb):
    src = adj_indices[1]
    dst = adj_indices[0]
    e0lo = jnp.concatenate([user_emb[:, :_H], item_emb[:, :_H]], axis=0)
    e0hi = jnp.concatenate([user_emb[:, _H:], item_emb[:, _H:]], axis=0)
    layer = _make_layer()
    e1lo, e1hi = layer(src, dst, adj_values, e0lo, e0hi)
    e2lo, e2hi = layer(src, dst, adj_values, e1lo, e1hi)
    e3lo, e3hi = layer(src, dst, adj_values, e2lo, e2hi)
    flo, fhi = _make_mean()(e0lo, e1lo, e2lo, e3lo, e0hi, e1hi, e2hi, e3hi)
    users = jnp.concatenate([flo[:_N_USERS], fhi[:_N_USERS]], axis=1)
    items = jnp.concatenate([flo[_N_USERS:], fhi[_N_USERS:]], axis=1)
    return users, items


# speedup vs baseline: 3.4552x; 3.4552x over previous
import functools

import jax
import jax.numpy as jnp
from jax import lax
from jax.experimental import pallas as pl
from jax.experimental.pallas import tpu as pltpu
from jax.experimental.pallas import tpu_sc as plsc

_N_USERS = 25000
_N_ITEMS = 25000
_N = _N_USERS + _N_ITEMS
_H = 32
_E = 800000
_C = 128
_CHUNKS = _E // _C
_NSUB = 16
_FULL = _CHUNKS // _NSUB
_REM = _CHUNKS - _FULL * _NSUB
_RPT = 3200
_WB = 200


def _layer_body(src_hbm, dst_hbm, vals_hbm, elo_hbm, ehi_hbm,
                outlo_hbm, outhi_hbm,
                acc, src_v, dst_v, vals_v, rows_v, zb, sem):
    cid = lax.axis_index("c")
    sid = lax.axis_index("s")

    zeros16 = jnp.zeros((16,), jnp.float32)

    def zb_zero(i, carry):
        zb[i, pl.ds(0, 16)] = zeros16
        zb[i, pl.ds(16, 16)] = zeros16
        return carry

    lax.fori_loop(0, _WB, zb_zero, 0)
    nwb = jnp.where(sid < _NSUB - 1, _RPT // _WB, (_N - (_NSUB - 1) * _RPT) // _WB)

    def acc_zero(j, carry):
        pltpu.sync_copy(zb, acc.at[pl.ds(sid * _RPT + j * _WB, _WB)])
        return carry

    lax.fori_loop(0, nwb, acc_zero, 0)
    plsc.subcore_barrier()

    def run_half(e_hbm, out_hbm):
        nchunks = _FULL + jnp.where(sid < _REM, 1, 0)

        def chunk_body(i, carry):
            base = (i * _NSUB + sid) * _C
            pltpu.sync_copy(src_hbm.at[pl.ds(base, _C)], src_v)
            pltpu.sync_copy(dst_hbm.at[pl.ds(base, _C)], dst_v)
            pltpu.sync_copy(vals_hbm.at[pl.ds(base, _C)], vals_v)
            pltpu.async_copy(e_hbm.at[src_v], rows_v, sem).wait()
            gdn = lax.GatherDimensionNumbers(
                offset_dims=(), collapsed_slice_dims=(0,), start_index_map=(0,))
            for g in range(_C // 16):
                vals16 = vals_v[pl.ds(g * 16, 16)]
                for r in range(16):
                    v16 = lax.gather(
                        vals16, jnp.full((16, 1), r, jnp.int32), gdn, (1,),
                        mode=lax.GatherScatterMode.PROMISE_IN_BOUNDS)
                    row = g * 16 + r
                    rows_v[row, pl.ds(0, 16)] = rows_v[row, pl.ds(0, 16)] * v16
                    rows_v[row, pl.ds(16, 16)] = rows_v[row, pl.ds(16, 16)] * v16
            pltpu.sync_copy(rows_v, acc.at[dst_v], add=True)
            return carry

        lax.fori_loop(0, nchunks, chunk_body, 0)
        plsc.subcore_barrier()

        def wb(j, carry):
            off = sid * _RPT + j * _WB
            pltpu.sync_copy(acc.at[pl.ds(off, _WB)], zb)
            pltpu.sync_copy(zb, out_hbm.at[pl.ds(off, _WB)])
            return carry

        lax.fori_loop(0, nwb, wb, 0)

    @pl.when(cid == 0)
    def _():
        run_half(elo_hbm, outlo_hbm)

    @pl.when(cid == 1)
    def _():
        run_half(ehi_hbm, outhi_hbm)


@functools.cache
def _make_layer():
    mesh = plsc.VectorSubcoreMesh(core_axis_name="c", subcore_axis_name="s")
    return pl.kernel(
        _layer_body,
        out_type=[jax.ShapeDtypeStruct((_N, _H), jnp.float32)] * 2,
        mesh=mesh,
        scratch_types=[
            pltpu.VMEM_SHARED((_N, _H), jnp.float32),
            pltpu.VMEM((_C,), jnp.int32),
            pltpu.VMEM((_C,), jnp.int32),
            pltpu.VMEM((_C,), jnp.float32),
            pltpu.VMEM((_C, _H), jnp.float32),
            pltpu.VMEM((_WB, _H), jnp.float32),
            pltpu.SemaphoreType.DMA,
        ],
        compiler_params=pltpu.CompilerParams(use_tc_tiling_on_sc=False),
    )


def _mean_body(a0, a1, a2, a3, b0, b1, b2, b3, olo, ohi):
    olo[...] = (a0[...] + a1[...] + a2[...] + a3[...]) * 0.25
    ohi[...] = (b0[...] + b1[...] + b2[...] + b3[...]) * 0.25


_BLK = 400


@functools.cache
def _make_mean():
    spec = pl.BlockSpec((_BLK, _H), lambda i: (i, 0))
    return pl.pallas_call(
        _mean_body,
        grid=(_N // _BLK,),
        in_specs=[spec] * 8,
        out_specs=[spec] * 2,
        out_shape=[jax.ShapeDtypeStruct((_N, _H), jnp.float32)] * 2,
    )


def kernel(adj_indices, adj_values, user_emb, item_emb):
    src = adj_indices[1]
    dst = adj_indices[0]
    e0lo = jnp.concatenate([user_emb[:, :_H], item_emb[:, :_H]], axis=0)
    e0hi = jnp.concatenate([user_emb[:, _H:], item_emb[:, _H:]], axis=0)
    layer = _make_layer()
    e1lo, e1hi = layer(src, dst, adj_values, e0lo, e0hi)
    e2lo, e2hi = layer(src, dst, adj_values, e1lo, e1hi)
    e3lo, e3hi = layer(src, dst, adj_values, e2lo, e2hi)
    flo, fhi = _make_mean()(e0lo, e1lo, e2lo, e3lo, e0hi, e1hi, e2hi, e3hi)
    users = jnp.concatenate([flo[:_N_USERS], fhi[:_N_USERS]], axis=1)
    items = jnp.concatenate([flo[_N_USERS:], fhi[_N_USERS:]], axis=1)
    return users, items


# speedup vs baseline: 8.9155x; 2.5803x over previous
import functools

import jax
import jax.numpy as jnp
from jax import lax
from jax.experimental import pallas as pl
from jax.experimental.pallas import tpu as pltpu
from jax.experimental.pallas import tpu_sc as plsc

_N_USERS = 25000
_N_ITEMS = 25000
_N = _N_USERS + _N_ITEMS
_H = 32
_E = 800000
_C = 128
_NSUB = 16
_TCH = 392
_CHUNKS = _TCH * _NSUB
_EP = _CHUNKS * _C
_GC = 28
_NG = _TCH // _GC
_RPT = 3200
_WB = 200

_GDN = lax.GatherDimensionNumbers(
    offset_dims=(), collapsed_slice_dims=(0,), start_index_map=(0,))


def _scale_chunk(rows, vals_blk, j):
    for g in range(_C // 16):
        vals16 = vals_blk[j, pl.ds(g * 16, 16)]
        for r in range(16):
            v16 = lax.gather(vals16, jnp.full((16, 1), r, jnp.int32), _GDN,
                             (1,), mode=lax.GatherScatterMode.PROMISE_IN_BOUNDS)
            row = g * 16 + r
            rows[row, pl.ds(0, 16)] = rows[row, pl.ds(0, 16)] * v16
            rows[row, pl.ds(16, 16)] = rows[row, pl.ds(16, 16)] * v16


def _layer_body(src_hbm, dst_hbm, vals_hbm, elo_hbm, ehi_hbm,
                outlo_hbm, outhi_hbm,
                acc, src_blk, dst_blk, vals_blk, rows0, rows1, zb,
                g0, g1, s0, s1):
    cid = lax.axis_index("c")
    sid = lax.axis_index("s")

    zeros16 = jnp.zeros((16,), jnp.float32)

    def zb_zero(i, carry):
        zb[i, pl.ds(0, 16)] = zeros16
        zb[i, pl.ds(16, 16)] = zeros16
        return carry

    lax.fori_loop(0, _WB, zb_zero, 0)
    nwb = jnp.where(sid < _NSUB - 1, _RPT // _WB, (_N - (_NSUB - 1) * _RPT) // _WB)

    def acc_zero(j, carry):
        pltpu.sync_copy(zb, acc.at[pl.ds(sid * _RPT + j * _WB, _WB)])
        return carry

    lax.fori_loop(0, nwb, acc_zero, 0)
    plsc.subcore_barrier()

    def run_half(e_hbm, out_hbm):
        def group_body(g, carry):
            grow = sid * _TCH + g * _GC
            pltpu.sync_copy(src_hbm.at[pl.ds(grow, _GC)], src_blk)
            pltpu.sync_copy(dst_hbm.at[pl.ds(grow, _GC)], dst_blk)
            pltpu.sync_copy(vals_hbm.at[pl.ds(grow, _GC)], vals_blk)
            pltpu.async_copy(e_hbm.at[src_blk.at[0]], rows0, g0)
            pltpu.async_copy(e_hbm.at[src_blk.at[1]], rows1, g1)

            def pair_body(p, c2):
                j0 = 2 * p
                j1 = 2 * p + 1
                pltpu.make_async_copy(e_hbm.at[src_blk.at[j0]], rows0, g0).wait()
                _scale_chunk(rows0, vals_blk, j0)
                pltpu.async_copy(rows0, acc.at[dst_blk.at[j0]], s0, add=True)
                pltpu.make_async_copy(e_hbm.at[src_blk.at[j1]], rows1, g1).wait()
                _scale_chunk(rows1, vals_blk, j1)
                pltpu.async_copy(rows1, acc.at[dst_blk.at[j1]], s1, add=True)

                @pl.when(p < _GC // 2 - 1)
                def _():
                    pltpu.make_async_copy(rows0, acc.at[dst_blk.at[j0]], s0).wait()
                    pltpu.async_copy(e_hbm.at[src_blk.at[j0 + 2]], rows0, g0)
                    pltpu.make_async_copy(rows1, acc.at[dst_blk.at[j1]], s1).wait()
                    pltpu.async_copy(e_hbm.at[src_blk.at[j1 + 2]], rows1, g1)

                return c2

            lax.fori_loop(0, _GC // 2, pair_body, 0)
            pltpu.make_async_copy(rows0, acc.at[dst_blk.at[0]], s0).wait()
            pltpu.make_async_copy(rows1, acc.at[dst_blk.at[1]], s1).wait()
            return carry

        lax.fori_loop(0, _NG, group_body, 0)
        plsc.subcore_barrier()

        def wb(j, carry):
            off = sid * _RPT + j * _WB
            pltpu.sync_copy(acc.at[pl.ds(off, _WB)], zb)
            pltpu.sync_copy(zb, out_hbm.at[pl.ds(off, _WB)])
            return carry

        lax.fori_loop(0, nwb, wb, 0)

    @pl.when(cid == 0)
    def _():
        run_half(elo_hbm, outlo_hbm)

    @pl.when(cid == 1)
    def _():
        run_half(ehi_hbm, outhi_hbm)


@functools.cache
def _make_layer():
    mesh = plsc.VectorSubcoreMesh(core_axis_name="c", subcore_axis_name="s")
    return pl.kernel(
        _layer_body,
        out_type=[jax.ShapeDtypeStruct((_N, _H), jnp.float32)] * 2,
        mesh=mesh,
        scratch_types=[
            pltpu.VMEM_SHARED((_N, _H), jnp.float32),
            pltpu.VMEM((_GC, _C), jnp.int32),
            pltpu.VMEM((_GC, _C), jnp.int32),
            pltpu.VMEM((_GC, _C), jnp.float32),
            pltpu.VMEM((_C, _H), jnp.float32),
            pltpu.VMEM((_C, _H), jnp.float32),
            pltpu.VMEM((_WB, _H), jnp.float32),
            pltpu.SemaphoreType.DMA,
            pltpu.SemaphoreType.DMA,
            pltpu.SemaphoreType.DMA,
            pltpu.SemaphoreType.DMA,
        ],
        compiler_params=pltpu.CompilerParams(use_tc_tiling_on_sc=False),
    )


def _mean_body(a0, a1, a2, a3, b0, b1, b2, b3, olo, ohi):
    olo[...] = (a0[...] + a1[...] + a2[...] + a3[...]) * 0.25
    ohi[...] = (b0[...] + b1[...] + b2[...] + b3[...]) * 0.25


_BLK = 400


@functools.cache
def _make_mean():
    spec = pl.BlockSpec((_BLK, _H), lambda i: (i, 0))
    return pl.pallas_call(
        _mean_body,
        grid=(_N // _BLK,),
        in_specs=[spec] * 8,
        out_specs=[spec] * 2,
        out_shape=[jax.ShapeDtypeStruct((_N, _H), jnp.float32)] * 2,
    )


def kernel(adj_indices, adj_values, user_emb, item_emb):
    pad = _EP - _E
    src = jnp.concatenate([adj_indices[1], jnp.zeros((pad,), jnp.int32)])
    dst = jnp.concatenate([adj_indices[0], jnp.zeros((pad,), jnp.int32)])
    vals = jnp.concatenate([adj_values, jnp.zeros((pad,), jnp.float32)])
    src2 = src.reshape(_CHUNKS, _C)
    dst2 = dst.reshape(_CHUNKS, _C)
    vals2 = vals.reshape(_CHUNKS, _C)
    e0lo = jnp.concatenate([user_emb[:, :_H], item_emb[:, :_H]], axis=0)
    e0hi = jnp.concatenate([user_emb[:, _H:], item_emb[:, _H:]], axis=0)
    layer = _make_layer()
    e1lo, e1hi = layer(src2, dst2, vals2, e0lo, e0hi)
    e2lo, e2hi = layer(src2, dst2, vals2, e1lo, e1hi)
    e3lo, e3hi = layer(src2, dst2, vals2, e2lo, e2hi)
    flo, fhi = _make_mean()(e0lo, e1lo, e2lo, e3lo, e0hi, e1hi, e2hi, e3hi)
    users = jnp.concatenate([flo[:_N_USERS], fhi[:_N_USERS]], axis=1)
    items = jnp.concatenate([flo[_N_USERS:], fhi[_N_USERS:]], axis=1)
    return users, items


# speedup vs baseline: 11.2787x; 1.2651x over previous
import functools

import jax
import jax.numpy as jnp
from jax import lax
from jax.experimental import pallas as pl
from jax.experimental.pallas import tpu as pltpu
from jax.experimental.pallas import tpu_sc as plsc

_N_USERS = 25000
_N_ITEMS = 25000
_N = _N_USERS + _N_ITEMS
_H = 32
_E = 800000
_C = 128
_NSUB = 16
_TCH = 392
_CHUNKS = _TCH * _NSUB
_EP = _CHUNKS * _C
_GC = 28
_NG = _TCH // _GC
_R = 4
_D = 2
_RPT = _N // _NSUB
_ZB = 125

_GDN = lax.GatherDimensionNumbers(
    offset_dims=(), collapsed_slice_dims=(0,), start_index_map=(0,))


def _scale_chunk(rows, vals_blk, j):
    for g in range(_C // 16):
        vals16 = vals_blk[j, pl.ds(g * 16, 16)]
        for r in range(16):
            v16 = lax.gather(vals16, jnp.full((16, 1), r, jnp.int32), _GDN,
                             (1,), mode=lax.GatherScatterMode.PROMISE_IN_BOUNDS)
            row = g * 16 + r
            rows[row, pl.ds(0, 16)] = rows[row, pl.ds(0, 16)] * v16
            rows[row, pl.ds(16, 16)] = rows[row, pl.ds(16, 16)] * v16


def _layer_body(src_hbm, dst_hbm, vals_hbm, elo_hbm, ehi_hbm,
                outlo_hbm, outhi_hbm,
                acc, src_blk, dst_blk, vals_blk, rows, gsem, ssem):
    cid = lax.axis_index("c")
    sid = lax.axis_index("s")

    zeros16 = jnp.zeros((16,), jnp.float32)

    def rb_zero(i, carry):
        rows[0][i, pl.ds(0, 16)] = zeros16
        rows[0][i, pl.ds(16, 16)] = zeros16
        return carry

    lax.fori_loop(0, _C, rb_zero, 0)

    def acc_zero(j, carry):
        pltpu.sync_copy(rows[0].at[pl.ds(0, _ZB)],
                        acc.at[pl.ds(sid * _RPT + j * _ZB, _ZB)])
        return carry

    lax.fori_loop(0, _RPT // _ZB, acc_zero, 0)
    plsc.subcore_barrier()

    def run_half(e_hbm, out_hbm):
        def group_body(g, carry):
            grow = sid * _TCH + g * _GC
            pltpu.sync_copy(src_hbm.at[pl.ds(grow, _GC)], src_blk)
            pltpu.sync_copy(dst_hbm.at[pl.ds(grow, _GC)], dst_blk)
            pltpu.sync_copy(vals_hbm.at[pl.ds(grow, _GC)], vals_blk)
            for b in range(_D):
                pltpu.async_copy(e_hbm.at[src_blk.at[b]], rows[b], gsem[b])

            def ring_body(q, c2):
                for b in range(_R):
                    j = q * _R + b
                    jn = j + _D
                    bn = (b + _D) % _R

                    @pl.when(jnp.logical_and(jn >= _R, jn < _GC))
                    def _(jn=jn, bn=bn):
                        pltpu.make_async_copy(
                            rows[bn], acc.at[dst_blk.at[0]], ssem[bn]).wait()
                        pltpu.async_copy(
                            e_hbm.at[src_blk.at[jn]], rows[bn], gsem[bn])

                    @pl.when(jn < _R)
                    def _(jn=jn, bn=bn):
                        pltpu.async_copy(
                            e_hbm.at[src_blk.at[jn]], rows[bn], gsem[bn])

                    pltpu.make_async_copy(
                        e_hbm.at[src_blk.at[j]], rows[b], gsem[b]).wait()
                    _scale_chunk(rows[b], vals_blk, j)
                    pltpu.async_copy(rows[b], acc.at[dst_blk.at[j]], ssem[b],
                                     add=True)
                return c2

            lax.fori_loop(0, _GC // _R, ring_body, 0)
            for b in range(_R):
                pltpu.make_async_copy(rows[b], acc.at[dst_blk.at[0]], ssem[b]).wait()
            return carry

        lax.fori_loop(0, _NG, group_body, 0)
        plsc.subcore_barrier()
        off = sid * _RPT
        pltpu.sync_copy(acc.at[pl.ds(off, _RPT)], out_hbm.at[pl.ds(off, _RPT)])

    @pl.when(cid == 0)
    def _():
        run_half(elo_hbm, outlo_hbm)

    @pl.when(cid == 1)
    def _():
        run_half(ehi_hbm, outhi_hbm)


@functools.cache
def _make_layer():
    mesh = plsc.VectorSubcoreMesh(core_axis_name="c", subcore_axis_name="s")
    return pl.kernel(
        _layer_body,
        out_type=[jax.ShapeDtypeStruct((_N, _H), jnp.float32)] * 2,
        mesh=mesh,
        scratch_types=[
            pltpu.VMEM_SHARED((_N, _H), jnp.float32),
            pltpu.VMEM((_GC, _C), jnp.int32),
            pltpu.VMEM((_GC, _C), jnp.int32),
            pltpu.VMEM((_GC, _C), jnp.float32),
            [pltpu.VMEM((_C, _H), jnp.float32)] * _R,
            [pltpu.SemaphoreType.DMA] * _R,
            [pltpu.SemaphoreType.DMA] * _R,
        ],
        compiler_params=pltpu.CompilerParams(use_tc_tiling_on_sc=False),
    )


def _mean_body(a0, a1, a2, a3, b0, b1, b2, b3, olo, ohi):
    olo[...] = (a0[...] + a1[...] + a2[...] + a3[...]) * 0.25
    ohi[...] = (b0[...] + b1[...] + b2[...] + b3[...]) * 0.25


_BLK = 400


@functools.cache
def _make_mean():
    spec = pl.BlockSpec((_BLK, _H), lambda i: (i, 0))
    return pl.pallas_call(
        _mean_body,
        grid=(_N // _BLK,),
        in_specs=[spec] * 8,
        out_specs=[spec] * 2,
        out_shape=[jax.ShapeDtypeStruct((_N, _H), jnp.float32)] * 2,
    )


def kernel(adj_indices, adj_values, user_emb, item_emb):
    pad = _EP - _E
    src = jnp.concatenate([adj_indices[1], jnp.zeros((pad,), jnp.int32)])
    dst = jnp.concatenate([adj_indices[0], jnp.zeros((pad,), jnp.int32)])
    vals = jnp.concatenate([adj_values, jnp.zeros((pad,), jnp.float32)])
    src2 = src.reshape(_CHUNKS, _C)
    dst2 = dst.reshape(_CHUNKS, _C)
    vals2 = vals.reshape(_CHUNKS, _C)
    e0lo = jnp.concatenate([user_emb[:, :_H], item_emb[:, :_H]], axis=0)
    e0hi = jnp.concatenate([user_emb[:, _H:], item_emb[:, _H:]], axis=0)
    layer = _make_layer()
    e1lo, e1hi = layer(src2, dst2, vals2, e0lo, e0hi)
    e2lo, e2hi = layer(src2, dst2, vals2, e1lo, e1hi)
    e3lo, e3hi = layer(src2, dst2, vals2, e2lo, e2hi)
    flo, fhi = _make_mean()(e0lo, e1lo, e2lo, e3lo, e0hi, e1hi, e2hi, e3hi)
    users = jnp.concatenate([flo[:_N_USERS], fhi[:_N_USERS]], axis=1)
    items = jnp.concatenate([flo[_N_USERS:], fhi[_N_USERS:]], axis=1)
    return users, items
